# baseline (speedup 1.0000x reference)
import functools

import jax
import jax.numpy as jnp
from jax import lax
from jax.experimental import pallas as pl
from jax.experimental.pallas import tpu as pltpu
from jax.experimental.pallas import tpu_sc as plsc

_NUM_AA = 20
_L = 200
_D = 128
_NC, _NS = 2, 16
_NW = _NC * _NS
_LANES = 16
_CHUNK = 128
_NBUF = 4


def _prep_body(aa_ref, pos_ref, len_ref, c_ref, mask_ref):
    c_ref[...] = pos_ref[...][:, None, :] + aa_ref[...][None, :, :]
    pos_iota = lax.broadcasted_iota(jnp.int32, mask_ref.shape, 1)
    mask_ref[...] = pos_iota >= len_ref[...][:, None]


def kernel(aa_indices, seq_lengths, aa_table, pos_table):
    B, Lc = aa_indices.shape
    seq_lengths = seq_lengths.astype(jnp.int32)

    c, mask = pl.pallas_call(
        _prep_body,
        out_shape=(
            jax.ShapeDtypeStruct((_L, _NUM_AA, _D), jnp.float32),
            jax.ShapeDtypeStruct((B, Lc), jnp.bool_),
        ),
    )(aa_table, pos_table, seq_lengths)
    c2 = c.reshape(_L * _NUM_AA, _D)

    R = B * Lc
    rows_per_w = R // _NW
    n_chunks = rows_per_w // _CHUNK
    idx3 = aa_indices.astype(jnp.int32).reshape(_NW, n_chunks, _CHUNK)

    mesh = plsc.VectorSubcoreMesh(core_axis_name="c", subcore_axis_name="s")

    @functools.partial(
        pl.kernel,
        out_type=jax.ShapeDtypeStruct((R, _D), jnp.float32),
        mesh=mesh,
        scratch_types=[
            pltpu.VMEM((n_chunks, _CHUNK), jnp.int32),
            pltpu.VMEM((2, 2 * _CHUNK, _D), jnp.float32),
            pltpu.VMEM_SHARED((_L * _NUM_AA, _D), jnp.float32),
            [pltpu.SemaphoreType.DMA] * 2,
            [pltpu.SemaphoreType.DMA] * 2,
        ],
    )
    def sc_gather(idx_hbm, c_hbm, out_hbm, flat_v, bufs, c_sh, gsems, wsems):
        wid = lax.axis_index("s") * _NC + lax.axis_index("c")
        base = wid * rows_per_w

        @pl.when(lax.axis_index("s") == 0)
        def _load_table():
            pltpu.sync_copy(c_hbm, c_sh)

        plsc.subcore_barrier()

        pltpu.sync_copy(idx_hbm.at[wid], flat_v)

        def compute_flat(k):
            m = lax.rem(k * _CHUNK, _L)
            for g in range(_CHUNK // _LANES):
                p = m + g * _LANES + lax.iota(jnp.int32, _LANES)
                p = jnp.where(p >= _L, p - _L, p)
                sl = pl.ds(g * _LANES, _LANES)
                flat_v[k, sl] = flat_v[k, sl] + p * _NUM_AA

        def start_gather_pair(t, half):
            pltpu.async_copy(c_sh.at[flat_v.at[2 * t]],
                             bufs.at[half, pl.ds(0, _CHUNK)], gsems[half])
            pltpu.async_copy(c_sh.at[flat_v.at[2 * t + 1]],
                             bufs.at[half, pl.ds(_CHUNK, _CHUNK)],
                             gsems[half])

        def wait_gather_pair(half):
            pltpu.make_async_copy(
                c_sh.at[flat_v.at[0]], bufs.at[half], gsems[half]).wait()

        def start_write_pair(t, half):
            pltpu.async_copy(
                bufs.at[half],
                out_hbm.at[pl.ds(base + t * 2 * _CHUNK, 2 * _CHUNK)],
                wsems[half])

        def wait_write_pair(half):
            pltpu.make_async_copy(
                bufs.at[half], out_hbm.at[pl.ds(base, 2 * _CHUNK)],
                wsems[half]).wait()

        n_pairs = n_chunks // 2
        compute_flat(0)
        compute_flat(1)
        start_gather_pair(0, 0)
        compute_flat(2)
        compute_flat(3)
        start_gather_pair(1, 1)
        wait_gather_pair(0)
        start_write_pair(0, 0)

        def main_body(j, carry):
            for hn in range(2):
                t = 1 + j * 2 + hn
                h = (1 + hn) % 2
                compute_flat(2 * t + 2)
                compute_flat(2 * t + 3)
                wait_write_pair(1 - h)
                start_gather_pair(t + 1, 1 - h)
                wait_gather_pair(h)
                start_write_pair(t, h)
            return carry

        lax.fori_loop(0, (n_pairs - 2) // 2, main_body, 0)

        wait_write_pair(0)
        wait_gather_pair(1)
        start_write_pair(n_pairs - 1, 1)
        wait_write_pair(1)

    out = sc_gather(idx3, c2)
    return out.reshape(B, Lc, _D), mask


# speedup vs baseline: 1.0259x; 1.0259x over previous
import functools

import jax
import jax.numpy as jnp
from jax import lax
from jax.experimental import pallas as pl
from jax.experimental.pallas import tpu as pltpu
from jax.experimental.pallas import tpu_sc as plsc

_NUM_AA = 20
_L = 200
_D = 128
_NC, _NS = 2, 16
_NW = _NC * _NS
_LANES = 16
_CHUNK = 128
_NBUF = 4


def _prep_body(aa_ref, pos_ref, len_ref, c_ref, mask_ref):
    c_ref[...] = pos_ref[...][:, None, :] + aa_ref[...][None, :, :]
    pos_iota = lax.broadcasted_iota(jnp.int32, mask_ref.shape, 1)
    mask_ref[...] = pos_iota >= len_ref[...][:, None]


def kernel(aa_indices, seq_lengths, aa_table, pos_table):
    B, Lc = aa_indices.shape
    seq_lengths = seq_lengths.astype(jnp.int32)

    c, mask = pl.pallas_call(
        _prep_body,
        out_shape=(
            jax.ShapeDtypeStruct((_L, _NUM_AA, _D), jnp.float32),
            jax.ShapeDtypeStruct((B, Lc), jnp.bool_),
        ),
    )(aa_table, pos_table, seq_lengths)
    c2 = c.reshape(_L * _NUM_AA, _D)

    R = B * Lc
    rows_per_w = R // _NW
    n_chunks = rows_per_w // _CHUNK
    idx3 = aa_indices.astype(jnp.int32).reshape(_NW, n_chunks, _CHUNK)

    mesh = plsc.VectorSubcoreMesh(core_axis_name="c", subcore_axis_name="s")

    @functools.partial(
        pl.kernel,
        out_type=jax.ShapeDtypeStruct((R, _D), jnp.float32),
        mesh=mesh,
        scratch_types=[
            pltpu.VMEM((n_chunks, _CHUNK), jnp.int32),
            pltpu.VMEM((_NBUF, _CHUNK, _D), jnp.float32),
            pltpu.VMEM_SHARED((_L * _NUM_AA, _D), jnp.float32),
            [pltpu.SemaphoreType.DMA] * _NBUF,
            [pltpu.SemaphoreType.DMA] * _NBUF,
            pltpu.SemaphoreType.DMA,
        ],
    )
    def sc_gather(idx_hbm, c_hbm, out_hbm, flat_v, bufs, c_sh, gsems, wsems,
                  isem):
        sid = lax.axis_index("s")
        wid = sid * _NC + lax.axis_index("c")
        base = wid * rows_per_w

        idx_cp = pltpu.async_copy(idx_hbm.at[wid], flat_v, isem)

        @pl.when(sid < _NS - 1)
        def _load_slice():
            pltpu.sync_copy(c_hbm.at[pl.ds(sid * 256, 256)],
                            c_sh.at[pl.ds(sid * 256, 256)])

        @pl.when(sid == _NS - 1)
        def _load_last():
            pltpu.sync_copy(c_hbm.at[pl.ds(sid * 256, 160)],
                            c_sh.at[pl.ds(sid * 256, 160)])

        plsc.subcore_barrier()
        idx_cp.wait()

        def compute_flat(k):
            m = lax.rem(k * _CHUNK, _L)
            for g in range(_CHUNK // _LANES):
                p = m + g * _LANES + lax.iota(jnp.int32, _LANES)
                p = jnp.where(p >= _L, p - _L, p)
                sl = pl.ds(g * _LANES, _LANES)
                flat_v[k, sl] = flat_v[k, sl] + p * _NUM_AA

        def start_gather(k, b):
            pltpu.async_copy(c_sh.at[flat_v.at[k]], bufs.at[b], gsems[b])

        def wait_gather(b):
            pltpu.make_async_copy(
                c_sh.at[flat_v.at[0]], bufs.at[b], gsems[b]).wait()

        def start_write(k, b):
            pltpu.async_copy(
                bufs.at[b], out_hbm.at[pl.ds(base + k * _CHUNK, _CHUNK)],
                wsems[b])

        def wait_write(b):
            pltpu.make_async_copy(
                bufs.at[b], out_hbm.at[pl.ds(base, _CHUNK)], wsems[b]).wait()

        compute_flat(0)
        compute_flat(1)
        start_gather(0, 0)
        start_gather(1, 1)
        for c in (0, 1):
            compute_flat(c + 2)
            start_gather(c + 2, (c + 2) % _NBUF)
            wait_gather(c % _NBUF)
            start_write(c, c % _NBUF)

        def main_body(j, carry):
            for b in range(_NBUF):
                c = 2 + j * _NBUF + b
                bc = (2 + b) % _NBUF
                bn = (bc + 2) % _NBUF
                compute_flat(c + 2)
                wait_write(bn)
                start_gather(c + 2, bn)
                wait_gather(bc)
                start_write(c, bc)
            return carry

        lax.fori_loop(0, (n_chunks - 4) // _NBUF, main_body, 0)

        for c in (n_chunks - 2, n_chunks - 1):
            bc = c % _NBUF
            wait_gather(bc)
            start_write(c, bc)
            wait_write((bc + 2) % _NBUF)
        for c in (n_chunks - 2, n_chunks - 1):
            wait_write(c % _NBUF)

    out = sc_gather(idx3, c2)
    return out.reshape(B, Lc, _D), mask


# speedup vs baseline: 1.0348x; 1.0086x over previous
import functools

import jax
import jax.numpy as jnp
from jax import lax
from jax.experimental import pallas as pl
from jax.experimental.pallas import tpu as pltpu
from jax.experimental.pallas import tpu_sc as plsc

_NUM_AA = 20
_L = 200
_D = 128
_NC, _NS = 2, 16
_NW = _NC * _NS
_LANES = 16
_CHUNK = 128
_NBUF = 4


def _prep_body(aa_ref, pos_ref, len_ref, c_ref, mask_ref):
    c_ref[...] = pos_ref[...][:, None, :] + aa_ref[...][None, :, :]
    pos_iota = lax.broadcasted_iota(jnp.int32, mask_ref.shape, 1)
    mask_ref[...] = (pos_iota >= len_ref[...][:, None]).astype(jnp.int8)


def kernel(aa_indices, seq_lengths, aa_table, pos_table):
    B, Lc = aa_indices.shape
    seq_lengths = seq_lengths.astype(jnp.int32)

    c, mask = pl.pallas_call(
        _prep_body,
        out_shape=(
            jax.ShapeDtypeStruct((_L, _NUM_AA, _D), jnp.float32),
            jax.ShapeDtypeStruct((B, Lc), jnp.int8),
        ),
    )(aa_table, pos_table, seq_lengths)
    mask = mask.astype(jnp.bool_)
    c2 = c.reshape(_L * _NUM_AA, _D)

    R = B * Lc
    rows_per_w = R // _NW
    n_chunks = rows_per_w // _CHUNK
    idx3 = aa_indices.astype(jnp.int32).reshape(_NW, n_chunks, _CHUNK)

    mesh = plsc.VectorSubcoreMesh(core_axis_name="c", subcore_axis_name="s")

    @functools.partial(
        pl.kernel,
        out_type=jax.ShapeDtypeStruct((R, _D), jnp.float32),
        mesh=mesh,
        scratch_types=[
            pltpu.VMEM((n_chunks, _CHUNK), jnp.int32),
            pltpu.VMEM((_NBUF, _CHUNK, _D), jnp.float32),
            pltpu.VMEM_SHARED((_L * _NUM_AA, _D), jnp.float32),
            [pltpu.SemaphoreType.DMA] * _NBUF,
            [pltpu.SemaphoreType.DMA] * _NBUF,
            pltpu.SemaphoreType.DMA,
        ],
    )
    def sc_gather(idx_hbm, c_hbm, out_hbm, flat_v, bufs, c_sh, gsems, wsems,
                  isem):
        sid = lax.axis_index("s")
        wid = sid * _NC + lax.axis_index("c")
        base = wid * rows_per_w

        idx_cp = pltpu.async_copy(idx_hbm.at[wid], flat_v, isem)

        @pl.when(sid < _NS - 1)
        def _load_slice():
            pltpu.sync_copy(c_hbm.at[pl.ds(sid * 256, 256)],
                            c_sh.at[pl.ds(sid * 256, 256)])

        @pl.when(sid == _NS - 1)
        def _load_last():
            pltpu.sync_copy(c_hbm.at[pl.ds(sid * 256, 160)],
                            c_sh.at[pl.ds(sid * 256, 160)])

        plsc.subcore_barrier()
        idx_cp.wait()

        def compute_flat(k):
            m = lax.rem(k * _CHUNK, _L)
            for g in range(_CHUNK // _LANES):
                p = m + g * _LANES + lax.iota(jnp.int32, _LANES)
                p = jnp.where(p >= _L, p - _L, p)
                sl = pl.ds(g * _LANES, _LANES)
                flat_v[k, sl] = flat_v[k, sl] + p * _NUM_AA

        def start_gather(k, b):
            pltpu.async_copy(c_sh.at[flat_v.at[k]], bufs.at[b], gsems[b])

        def wait_gather(b):
            pltpu.make_async_copy(
                c_sh.at[flat_v.at[0]], bufs.at[b], gsems[b]).wait()

        def start_write(k, b):
            pltpu.async_copy(
                bufs.at[b], out_hbm.at[pl.ds(base + k * _CHUNK, _CHUNK)],
                wsems[b])

        def wait_write(b):
            pltpu.make_async_copy(
                bufs.at[b], out_hbm.at[pl.ds(base, _CHUNK)], wsems[b]).wait()

        compute_flat(0)
        compute_flat(1)
        start_gather(0, 0)
        start_gather(1, 1)
        for c in (0, 1):
            compute_flat(c + 2)
            start_gather(c + 2, (c + 2) % _NBUF)
            wait_gather(c % _NBUF)
            start_write(c, c % _NBUF)

        def main_body(j, carry):
            for b in range(_NBUF):
                c = 2 + j * _NBUF + b
                bc = (2 + b) % _NBUF
                bn = (bc + 2) % _NBUF
                compute_flat(c + 2)
                wait_write(bn)
                start_gather(c + 2, bn)
                wait_gather(bc)
                start_write(c, bc)
            return carry

        lax.fori_loop(0, (n_chunks - 4) // _NBUF, main_body, 0)

        for c in (n_chunks - 2, n_chunks - 1):
            bc = c % _NBUF
            wait_gather(bc)
            start_write(c, bc)
            wait_write((bc + 2) % _NBUF)
        for c in (n_chunks - 2, n_chunks - 1):
            wait_write(c % _NBUF)

    out = sc_gather(idx3, c2)
    return out.reshape(B, Lc, _D), mask
